# baseline (speedup 1.0000x reference)
import functools

import jax
import jax.numpy as jnp
from jax import lax
from jax.experimental import pallas as pl
from jax.experimental.pallas import tpu as pltpu
from jax.experimental.pallas import tpu_sc as plsc

_info = plsc.get_sparse_core_info()
_NC, _NS = _info.num_cores, _info.num_subcores
_NW = _NC * _NS


@functools.lru_cache(maxsize=None)
def _make_gather5(V2: int, S: int, NB: int, L: int):
    assert NB == _NW
    n_idx = L * S
    mesh = plsc.VectorSubcoreMesh(core_axis_name="c", subcore_axis_name="s")

    @functools.partial(
        pl.kernel,
        mesh=mesh,
        compiler_params=pltpu.CompilerParams(
            use_tc_tiling_on_sc=False, needs_layout_passes=False),
        out_type=jax.ShapeDtypeStruct((S, 8, NB, 8, L), jnp.float32),
        scratch_types=[
            pltpu.VMEM((n_idx,), jnp.int32),
            pltpu.VMEM((S, 2 * L), jnp.int32),
            pltpu.VMEM((2 * L, 32), jnp.float32),
            pltpu.VMEM((2 * L, 32), jnp.float32),
            pltpu.VMEM((64, L), jnp.float32),
            pltpu.VMEM((64, L), jnp.float32),
            pltpu.SemaphoreType.DMA,
            pltpu.SemaphoreType.DMA,
            pltpu.SemaphoreType.DMA,
            pltpu.SemaphoreType.DMA,
        ],
    )
    def gather_kernel(w32_hbm, idx_hbm, out5_hbm, idx_v, idx2, bg0, bg1,
                      bt0, bt1, g0, g1, w0, w1):
        wid = lax.axis_index("s") * _NC + lax.axis_index("c")
        pltpu.sync_copy(idx_hbm.at[pl.ds(wid * n_idx, n_idx)], idx_v)
        iota = lax.iota(jnp.int32, 16)
        pos_base = [(iota + 16 * j) * S for j in range(L // 16)]
        col_base = [32 * j + 2 * iota for j in range(L // 16)]

        def build(s, carry):
            row = jnp.full((16,), s, jnp.int32)
            for j in range(L // 16):
                v = plsc.load_gather(idx_v, [pos_base[j] + s])
                a = v * 2
                plsc.store_scatter(idx2, [row, col_base[j]], a)
                plsc.store_scatter(idx2, [row, col_base[j] + 1], a + 1)
            return carry

        lax.fori_loop(0, S, build, 0)

        row_base = [2 * (iota + 16 * j) for j in range(L // 16)]

        bgs = (bg0, bg1)
        bts = (bt0, bt1)
        gsems = (g0, g1)
        wsems = (w0, w1)
        gcp = [None, None]
        wcp = [[], []]
        gcp[0] = pltpu.async_copy(w32_hbm.at[idx2.at[0]], bg0, g0)
        for s in range(S):
            p = s % 2
            if s + 1 < S:
                gcp[1 - p] = pltpu.async_copy(
                    w32_hbm.at[idx2.at[s + 1]], bgs[1 - p], gsems[1 - p])
            gcp[p].wait()
            for c in wcp[p]:
                c.wait()
            wcp[p] = []
            bg = bgs[p]
            bt = bts[p]

            def transpose(f, carry):
                c0 = f // 32
                col = jnp.full((16,), f % 32, jnp.int32)
                dst = bt.at[f]
                for j in range(L // 16):
                    v = plsc.load_gather(bg, [row_base[j] + c0, col])
                    dst[pl.ds(16 * j, 16)] = v
                return carry

            lax.fori_loop(0, 64, transpose, 0)
            for g in range(8):
                wcp[p].append(pltpu.async_copy(
                    bt.at[pl.ds(g * 8, 8)], out5_hbm.at[s, g, wid],
                    wsems[p]))
        for c in wcp[(S - 2) % 2]:
            c.wait()
        for c in wcp[(S - 1) % 2]:
            c.wait()

    return gather_kernel


def kernel(input, weight):
    NR, S = input.shape
    V, D = weight.shape
    L = 128
    NB = NR // L
    assert D == 64 and NR % L == 0
    idx = input.reshape(NR * S).astype(jnp.int32)
    w32 = weight.reshape(V * 2, 32)
    out5 = _make_gather5(V * 2, S, NB, L)(w32, idx)
    return out5.transpose(2, 4, 0, 1, 3).reshape(NR, S, D)
